# baseline (speedup 1.0000x reference)
import functools

import jax
import jax.numpy as jnp
from jax import lax
from jax.experimental import pallas as pl
from jax.experimental.pallas import tpu as pltpu
from jax.experimental.pallas import tpu_sc as plsc

N = 728
D = 64
E = 262144

_DOT = dict(preferred_element_type=jnp.float32, precision=lax.Precision.HIGHEST)
_DOTB = dict(preferred_element_type=jnp.float32, precision=lax.Precision.DEFAULT)

_NC = 2
_NS = 16
_NW = _NC * _NS
_L = 16
_HSZ = N * N
_EPW = E // _NW
_CHUNK = 128
_NCHUNK = _EPW // _CHUNK


_EHALF = _EPW // 2
_NROW = _EHALF // _CHUNK
_ZC = 4096
_DC = 2048
_NZFULL = (3 * _HSZ) // _ZC
_NDFULL = (3 * _HSZ) // _DC
_TAIL = 3 * _HSZ - _NZFULL * _ZC


def _hist_body(eg, ec, ef, out,
               e0a, e1a, e0b, e1b, idx_a, idx_b, ones_v, vbuf, hist_sh,
               zsem, esem, ssem, wsem):
    cid = lax.axis_index("c")
    sid = lax.axis_index("s")
    wid = sid * _NC + cid
    base = wid * _EPW
    edges = (eg, ec, ef)
    halves = [(g, h) for g in range(3) for h in range(2)]
    ebufs = ((e0a, e1a), (e0b, e1b))
    ibufs = (idx_a, idx_b)

    def fire_edges(t):
        g, h = halves[t]
        e0, e1 = ebufs[t % 2]
        off = base + h * _EHALF
        return (
            pltpu.async_copy(edges[g].at[0, pl.ds(off, _EHALF)], e0, esem),
            pltpu.async_copy(edges[g].at[1, pl.ds(off, _EHALF)], e1, esem),
        )

    ed = fire_edges(0)

    def zfill(i, _):
        for u in range(8):
            vbuf[pl.ds((i * 8 + u) * _L, _L)] = jnp.zeros((_L,), jnp.float32)
        return 0

    lax.fori_loop(0, _ZC // (8 * _L), zfill, 0)
    zdescs = []
    for j in range(_NZFULL // _NS):
        k = j * _NS + sid
        zdescs.append(pltpu.async_copy(
            vbuf.at[pl.ds(0, _ZC)], hist_sh.at[pl.ds(k * _ZC, _ZC)], zsem))

    @pl.when(sid < _NZFULL - (_NZFULL // _NS) * _NS)
    def _():
        k = (_NZFULL // _NS) * _NS + sid
        pltpu.sync_copy(vbuf.at[pl.ds(0, _ZC)],
                        hist_sh.at[pl.ds(k * _ZC, _ZC)])

    @pl.when(sid == 0)
    def _():
        pltpu.sync_copy(vbuf.at[pl.ds(0, _TAIL)],
                        hist_sh.at[pl.ds(_NZFULL * _ZC, _TAIL)])

    for i in range(_CHUNK // _L):
        ones_v[pl.ds(i * _L, _L)] = jnp.full((_L,), 1.0, jnp.float32)
    for d in zdescs:
        d.wait()

    plsc.subcore_barrier()

    sprev = []
    for t in range(6):
        g, _h = halves[t]
        e0, e1 = ebufs[t % 2]
        idx = ibufs[t % 2]
        ed[0].wait()
        ed[1].wait()
        if t + 1 < 6:
            ed = fire_edges(t + 1)
        g_off = g * _HSZ

        def idx_row(r, _, e0=e0, e1=e1, idx=idx, g_off=g_off):
            for c in range(_CHUNK // _L):
                o = c * _L
                v0 = e0[pl.ds(r * _CHUNK + o, _L)]
                v1 = e1[pl.ds(r * _CHUNK + o, _L)]
                idx[r, pl.ds(o, _L)] = v0 * N + v1 + g_off
            return 0

        lax.fori_loop(0, _NROW, idx_row, 0)
        for d in sprev:
            d.wait()
        sprev = [
            pltpu.async_copy(ones_v, hist_sh.at[idx.at[r]], ssem, add=True)
            for r in range(_NROW)
        ]
    for d in sprev:
        d.wait()

    plsc.subcore_barrier()

    out_off = cid * 3 * _HSZ
    wr = [None, None]
    for j in range(_NDFULL // _NS):
        p = j % 2
        k = j * _NS + sid
        if wr[p] is not None:
            wr[p].wait()
        pltpu.sync_copy(hist_sh.at[pl.ds(k * _DC, _DC)],
                        vbuf.at[pl.ds(p * _DC, _DC)])
        wr[p] = pltpu.async_copy(vbuf.at[pl.ds(p * _DC, _DC)],
                                 out.at[pl.ds(out_off + k * _DC, _DC)], wsem)
    for d in wr:
        if d is not None:
            d.wait()

    @pl.when(sid < _NDFULL - (_NDFULL // _NS) * _NS)
    def _():
        k = (_NDFULL // _NS) * _NS + sid
        pltpu.sync_copy(hist_sh.at[pl.ds(k * _DC, _DC)],
                        vbuf.at[pl.ds(0, _DC)])
        pltpu.sync_copy(vbuf.at[pl.ds(0, _DC)],
                        out.at[pl.ds(out_off + k * _DC, _DC)])

    @pl.when(sid == 0)
    def _():
        off = _NDFULL * _DC
        pltpu.sync_copy(hist_sh.at[pl.ds(off, _TAIL)],
                        vbuf.at[pl.ds(0, _TAIL)])
        pltpu.sync_copy(vbuf.at[pl.ds(0, _TAIL)],
                        out.at[pl.ds(out_off + off, _TAIL)])


def _histogram3(e_gua, e_cos, e_fun):
    run = pl.kernel(
        _hist_body,
        out_type=jax.ShapeDtypeStruct((_NC * 3 * _HSZ,), jnp.float32),
        mesh=plsc.VectorSubcoreMesh(core_axis_name="c", subcore_axis_name="s"),
        scratch_types=[
            pltpu.VMEM((_EHALF,), jnp.int32),
            pltpu.VMEM((_EHALF,), jnp.int32),
            pltpu.VMEM((_EHALF,), jnp.int32),
            pltpu.VMEM((_EHALF,), jnp.int32),
            pltpu.VMEM((_NROW, _CHUNK), jnp.int32),
            pltpu.VMEM((_NROW, _CHUNK), jnp.int32),
            pltpu.VMEM((_CHUNK,), jnp.float32),
            pltpu.VMEM((_ZC,), jnp.float32),
            pltpu.VMEM_SHARED((3 * _HSZ,), jnp.float32),
            pltpu.SemaphoreType.DMA,
            pltpu.SemaphoreType.DMA,
            pltpu.SemaphoreType.DMA,
            pltpu.SemaphoreType.DMA,
        ],
    )
    return run(e_gua, e_cos, e_fun).reshape(_NC, 3, N, N)


def _conv_body(cnt_ref, mi_ref, x_ref, W1_ref, b1_ref, W2_ref, b2_ref, out_ref):
    cnt = cnt_ref[0, 0] + cnt_ref[1, 0]
    C = cnt * mi_ref[0]
    deg = jnp.sum(C, axis=0, keepdims=True) + 1.0
    r = lax.rsqrt(deg)
    r = r * (1.5 - 0.5 * deg * r * r)
    dis = jnp.where(deg > 0, r, 0.0)

    def layer(x, W, b, x_is_nd):
        cdims = (((0,), (1,)), ((), ())) if x_is_nd else (((0,), (0,)), ((), ()))
        hT = lax.dot_general(W, x, cdims, **_DOT)
        HdT = hT * dis
        aggT = lax.dot_general(HdT, C, (((1,), (0,)), ((), ())), **_DOTB) + HdT
        return jnp.maximum(dis * aggT + b, 0.0)

    x1T = layer(x_ref[...], W1_ref[...], b1_ref[...], True)
    x2T = layer(x1T, W2_ref[...], b2_ref[...], False)
    out_ref[0:D, :] = x1T
    out_ref[D:2 * D, :] = x2T


def _convs_tc(cnt2, mi3, x, W1, b1c, W2, b2c):
    return pl.pallas_call(
        _conv_body,
        grid=(3,),
        in_specs=[
            pl.BlockSpec((2, 1, N, N), lambda i: (0, i, 0, 0)),
            pl.BlockSpec((1, N, N), lambda i: (i, 0, 0)),
            pl.BlockSpec((N, D), lambda i: (0, 0)),
            pl.BlockSpec((D, D), lambda i: (0, 0)),
            pl.BlockSpec((D, 1), lambda i: (0, 0)),
            pl.BlockSpec((D, D), lambda i: (0, 0)),
            pl.BlockSpec((D, 1), lambda i: (0, 0)),
        ],
        out_specs=pl.BlockSpec((2 * D, N), lambda i: (i, 0)),
        out_shape=jax.ShapeDtypeStruct((6 * D, N), jnp.float32),
    )(cnt2, mi3, x, W1, b1c, W2, b2c)


def _head_body(xm_ref, fc1_W_ref, fc1_b_ref, fc2_W_ref, fc2_b_ref,
               cnn_W_ref, cnn_b_ref, out_ref):
    xm = xm_ref[...]
    sums = [jnp.sum(xm[c]) for c in range(6)]
    att = jnp.stack(sums).reshape(1, 6) * (1.0 / (D * N))
    t1 = lax.dot_general(att, fc1_W_ref[...], (((1,), (1,)), ((), ())), **_DOT)
    t1 = jnp.maximum(t1 + fc1_b_ref[...], 0.0)
    t2 = lax.dot_general(t1, fc2_W_ref[...], (((1,), (1,)), ((), ())), **_DOT)
    t2 = t2 + fc2_b_ref[...]
    a = 1.0 / (1.0 + jnp.exp(-t2))
    coef = a * cnn_W_ref[...]
    acc = coef[0, 0] * xm[0]
    for c in range(1, 6):
        acc = acc + coef[0, c] * xm[c]
    out_ref[...] = acc + cnn_b_ref[0, 0]


def _head_tc(xm6, fc1_W, fc1_b, fc2_W, fc2_b, cnn_W, cnn_b):
    return pl.pallas_call(
        _head_body,
        out_shape=jax.ShapeDtypeStruct((D, N), jnp.float32),
    )(xm6, fc1_W, fc1_b, fc2_W, fc2_b, cnn_W, cnn_b)


def kernel(mi_gua, mi_gua_edges, mi_cos, mi_cos_edges, mi_fun, mi_fun_edges,
           mi_infeats, W1, b1, W2, b2, fc1_W, fc1_b, fc2_W, fc2_b, cnn_W, cnn_b):
    cnt2 = _histogram3(mi_gua_edges, mi_cos_edges, mi_fun_edges)
    mi3 = jnp.stack([mi_gua, mi_cos, mi_fun])
    XMT = _convs_tc(cnt2, mi3, mi_infeats,
                    W1, b1.reshape(D, 1), W2, b2.reshape(D, 1))
    xm6 = XMT.T.reshape(6, D, N)
    outT = _head_tc(xm6, fc1_W, fc1_b.reshape(1, 30),
                    fc2_W, fc2_b.reshape(1, 6),
                    cnn_W.reshape(1, 6), cnn_b.reshape(1, 1))
    return outT.T


# speedup vs baseline: 1.0015x; 1.0015x over previous
import functools

import jax
import jax.numpy as jnp
from jax import lax
from jax.experimental import pallas as pl
from jax.experimental.pallas import tpu as pltpu
from jax.experimental.pallas import tpu_sc as plsc

N = 728
D = 64
E = 262144

_DOT = dict(preferred_element_type=jnp.float32, precision=lax.Precision.HIGHEST)
_DOTB = dict(preferred_element_type=jnp.float32, precision=lax.Precision.DEFAULT)

_NC = 2
_NS = 16
_NW = _NC * _NS
_L = 16
_HSZ = N * N
_EPW = E // _NW
_CHUNK = 128
_NCHUNK = _EPW // _CHUNK


_EHALF = _EPW // 2
_NROW = _EHALF // _CHUNK
_ZC = 4096
_DC = 2048
_NZFULL = (3 * _HSZ) // _ZC
_NDFULL = (3 * _HSZ) // _DC
_TAIL = 3 * _HSZ - _NZFULL * _ZC


def _hist_body(eg, ec, ef, out,
               e0a, e1a, e0b, e1b, idx_a, idx_b, ones_v, vbuf, hist_sh,
               zsem, esem, ssem, wsem):
    cid = lax.axis_index("c")
    sid = lax.axis_index("s")
    wid = sid * _NC + cid
    base = wid * _EPW
    edges = (eg, ec, ef)
    halves = [(g, h) for g in range(3) for h in range(2)]
    ebufs = ((e0a, e1a), (e0b, e1b))
    ibufs = (idx_a, idx_b)

    def fire_edges(t):
        g, h = halves[t]
        e0, e1 = ebufs[t % 2]
        off = base + h * _EHALF
        return (
            pltpu.async_copy(edges[g].at[0, pl.ds(off, _EHALF)], e0, esem),
            pltpu.async_copy(edges[g].at[1, pl.ds(off, _EHALF)], e1, esem),
        )

    ed = fire_edges(0)

    sc0 = jax.named_scope("hist_zero")
    sc0.__enter__()

    def zfill(i, _):
        for u in range(8):
            vbuf[pl.ds((i * 8 + u) * _L, _L)] = jnp.zeros((_L,), jnp.float32)
        return 0

    lax.fori_loop(0, _ZC // (8 * _L), zfill, 0)
    zdescs = []
    for j in range(_NZFULL // _NS):
        k = j * _NS + sid
        zdescs.append(pltpu.async_copy(
            vbuf.at[pl.ds(0, _ZC)], hist_sh.at[pl.ds(k * _ZC, _ZC)], zsem))

    @pl.when(sid < _NZFULL - (_NZFULL // _NS) * _NS)
    def _():
        k = (_NZFULL // _NS) * _NS + sid
        pltpu.sync_copy(vbuf.at[pl.ds(0, _ZC)],
                        hist_sh.at[pl.ds(k * _ZC, _ZC)])

    @pl.when(sid == 0)
    def _():
        pltpu.sync_copy(vbuf.at[pl.ds(0, _TAIL)],
                        hist_sh.at[pl.ds(_NZFULL * _ZC, _TAIL)])

    for i in range(_CHUNK // _L):
        ones_v[pl.ds(i * _L, _L)] = jnp.full((_L,), 1.0, jnp.float32)
    for d in zdescs:
        d.wait()

    plsc.subcore_barrier()
    sc0.__exit__(None, None, None)
    sc1 = jax.named_scope("hist_scatter")
    sc1.__enter__()

    sprev = []
    for t in range(6):
        g, _h = halves[t]
        e0, e1 = ebufs[t % 2]
        idx = ibufs[t % 2]
        ed[0].wait()
        ed[1].wait()
        if t + 1 < 6:
            ed = fire_edges(t + 1)
        g_off = g * _HSZ

        def idx_row(r, _, e0=e0, e1=e1, idx=idx, g_off=g_off):
            for c in range(_CHUNK // _L):
                o = c * _L
                v0 = e0[pl.ds(r * _CHUNK + o, _L)]
                v1 = e1[pl.ds(r * _CHUNK + o, _L)]
                idx[r, pl.ds(o, _L)] = v0 * N + v1 + g_off
            return 0

        lax.fori_loop(0, _NROW, idx_row, 0)
        for d in sprev:
            d.wait()
        sprev = [
            pltpu.async_copy(ones_v, hist_sh.at[idx.at[r]], ssem, add=True)
            for r in range(_NROW)
        ]
    for d in sprev:
        d.wait()

    plsc.subcore_barrier()
    sc1.__exit__(None, None, None)
    sc2 = jax.named_scope("hist_drain")
    sc2.__enter__()

    out_off = cid * 3 * _HSZ
    wr = [None, None]
    for j in range(_NDFULL // _NS):
        p = j % 2
        k = j * _NS + sid
        if wr[p] is not None:
            wr[p].wait()
        pltpu.sync_copy(hist_sh.at[pl.ds(k * _DC, _DC)],
                        vbuf.at[pl.ds(p * _DC, _DC)])
        wr[p] = pltpu.async_copy(vbuf.at[pl.ds(p * _DC, _DC)],
                                 out.at[pl.ds(out_off + k * _DC, _DC)], wsem)
    for d in wr:
        if d is not None:
            d.wait()

    @pl.when(sid < _NDFULL - (_NDFULL // _NS) * _NS)
    def _():
        k = (_NDFULL // _NS) * _NS + sid
        pltpu.sync_copy(hist_sh.at[pl.ds(k * _DC, _DC)],
                        vbuf.at[pl.ds(0, _DC)])
        pltpu.sync_copy(vbuf.at[pl.ds(0, _DC)],
                        out.at[pl.ds(out_off + k * _DC, _DC)])

    @pl.when(sid == 0)
    def _():
        off = _NDFULL * _DC
        pltpu.sync_copy(hist_sh.at[pl.ds(off, _TAIL)],
                        vbuf.at[pl.ds(0, _TAIL)])
        pltpu.sync_copy(vbuf.at[pl.ds(0, _TAIL)],
                        out.at[pl.ds(out_off + off, _TAIL)])

    sc2.__exit__(None, None, None)


def _histogram3(e_gua, e_cos, e_fun):
    run = pl.kernel(
        _hist_body,
        out_type=jax.ShapeDtypeStruct((_NC * 3 * _HSZ,), jnp.float32),
        mesh=plsc.VectorSubcoreMesh(core_axis_name="c", subcore_axis_name="s"),
        scratch_types=[
            pltpu.VMEM((_EHALF,), jnp.int32),
            pltpu.VMEM((_EHALF,), jnp.int32),
            pltpu.VMEM((_EHALF,), jnp.int32),
            pltpu.VMEM((_EHALF,), jnp.int32),
            pltpu.VMEM((_NROW, _CHUNK), jnp.int32),
            pltpu.VMEM((_NROW, _CHUNK), jnp.int32),
            pltpu.VMEM((_CHUNK,), jnp.float32),
            pltpu.VMEM((_ZC,), jnp.float32),
            pltpu.VMEM_SHARED((3 * _HSZ,), jnp.float32),
            pltpu.SemaphoreType.DMA,
            pltpu.SemaphoreType.DMA,
            pltpu.SemaphoreType.DMA,
            pltpu.SemaphoreType.DMA,
        ],
    )
    return run(e_gua, e_cos, e_fun).reshape(_NC, 3, N, N)


def _conv_body(cnt_ref, mi_ref, x_ref, W1_ref, b1_ref, W2_ref, b2_ref, out_ref):
    cnt = cnt_ref[0, 0] + cnt_ref[1, 0]
    C = cnt * mi_ref[0]
    deg = jnp.sum(C, axis=0, keepdims=True) + 1.0
    r = lax.rsqrt(deg)
    r = r * (1.5 - 0.5 * deg * r * r)
    dis = jnp.where(deg > 0, r, 0.0)

    def layer(x, W, b, x_is_nd):
        cdims = (((0,), (1,)), ((), ())) if x_is_nd else (((0,), (0,)), ((), ()))
        hT = lax.dot_general(W, x, cdims, **_DOT)
        HdT = hT * dis
        aggT = lax.dot_general(HdT, C, (((1,), (0,)), ((), ())), **_DOTB) + HdT
        return jnp.maximum(dis * aggT + b, 0.0)

    x1T = layer(x_ref[...], W1_ref[...], b1_ref[...], True)
    x2T = layer(x1T, W2_ref[...], b2_ref[...], False)
    out_ref[0:D, :] = x1T
    out_ref[D:2 * D, :] = x2T


def _convs_tc(cnt2, mi3, x, W1, b1c, W2, b2c):
    return pl.pallas_call(
        _conv_body,
        grid=(3,),
        in_specs=[
            pl.BlockSpec((2, 1, N, N), lambda i: (0, i, 0, 0)),
            pl.BlockSpec((1, N, N), lambda i: (i, 0, 0)),
            pl.BlockSpec((N, D), lambda i: (0, 0)),
            pl.BlockSpec((D, D), lambda i: (0, 0)),
            pl.BlockSpec((D, 1), lambda i: (0, 0)),
            pl.BlockSpec((D, D), lambda i: (0, 0)),
            pl.BlockSpec((D, 1), lambda i: (0, 0)),
        ],
        out_specs=pl.BlockSpec((2 * D, N), lambda i: (i, 0)),
        out_shape=jax.ShapeDtypeStruct((6 * D, N), jnp.float32),
    )(cnt2, mi3, x, W1, b1c, W2, b2c)


def _head_body(xm_ref, fc1_W_ref, fc1_b_ref, fc2_W_ref, fc2_b_ref,
               cnn_W_ref, cnn_b_ref, out_ref):
    xm = xm_ref[...]
    sums = [jnp.sum(xm[c]) for c in range(6)]
    att = jnp.stack(sums).reshape(1, 6) * (1.0 / (D * N))
    t1 = lax.dot_general(att, fc1_W_ref[...], (((1,), (1,)), ((), ())), **_DOT)
    t1 = jnp.maximum(t1 + fc1_b_ref[...], 0.0)
    t2 = lax.dot_general(t1, fc2_W_ref[...], (((1,), (1,)), ((), ())), **_DOT)
    t2 = t2 + fc2_b_ref[...]
    a = 1.0 / (1.0 + jnp.exp(-t2))
    coef = a * cnn_W_ref[...]
    acc = coef[0, 0] * xm[0]
    for c in range(1, 6):
        acc = acc + coef[0, c] * xm[c]
    out_ref[...] = acc + cnn_b_ref[0, 0]


def _head_tc(xm6, fc1_W, fc1_b, fc2_W, fc2_b, cnn_W, cnn_b):
    return pl.pallas_call(
        _head_body,
        out_shape=jax.ShapeDtypeStruct((D, N), jnp.float32),
    )(xm6, fc1_W, fc1_b, fc2_W, fc2_b, cnn_W, cnn_b)


def kernel(mi_gua, mi_gua_edges, mi_cos, mi_cos_edges, mi_fun, mi_fun_edges,
           mi_infeats, W1, b1, W2, b2, fc1_W, fc1_b, fc2_W, fc2_b, cnn_W, cnn_b):
    cnt2 = _histogram3(mi_gua_edges, mi_cos_edges, mi_fun_edges)
    mi3 = jnp.stack([mi_gua, mi_cos, mi_fun])
    XMT = _convs_tc(cnt2, mi3, mi_infeats,
                    W1, b1.reshape(D, 1), W2, b2.reshape(D, 1))
    xm6 = XMT.T.reshape(6, D, N)
    outT = _head_tc(xm6, fc1_W, fc1_b.reshape(1, 30),
                    fc2_W, fc2_b.reshape(1, 6),
                    cnn_W.reshape(1, 6), cnn_b.reshape(1, 1))
    return outT.T


# speedup vs baseline: 1.1041x; 1.1025x over previous
import jax
import jax.numpy as jnp
from jax import lax
from jax.experimental import pallas as pl
from jax.experimental.pallas import tpu as pltpu
from jax.experimental.pallas import tpu_sc as plsc

N = 728
D = 64
E = 262144

_DOT = dict(preferred_element_type=jnp.float32, precision=lax.Precision.HIGHEST)
_DOTB = dict(preferred_element_type=jnp.float32, precision=lax.Precision.DEFAULT)

_NC = 2
_NS = 16
_NW = _NC * _NS
_L = 16

_W = 768
_PLN = N * _W
_PSZ = 3 * _PLN
_EPW = E // _NW
_ESH = _EPW // 4
_CHUNK = 128
_NROW = _ESH // _CHUNK
_ZC = 2048
_DC = 1024
_NZFULL = _PSZ // _ZC
_NDFULL = _PSZ // _DC


def _hist_body(eg, ec, ef, out,
               e0a, e1a, e0b, e1b, idx_a, idx_b, ones_v, vbuf, hist_sh,
               zsem, esem, ssem, wsem):
    cid = lax.axis_index("c")
    sid = lax.axis_index("s")
    wid = sid * _NC + cid
    base = wid * _EPW
    edges = (eg, ec, ef)
    steps = [(g, q) for g in range(3) for q in range(4)]
    ebufs = ((e0a, e1a), (e0b, e1b))
    ibufs = (idx_a, idx_b)

    def fire_edges(t):
        g, q = steps[t]
        e0, e1 = ebufs[t % 2]
        off = base + q * _ESH
        return (
            pltpu.async_copy(edges[g].at[0, pl.ds(off, _ESH)], e0, esem),
            pltpu.async_copy(edges[g].at[1, pl.ds(off, _ESH)], e1, esem),
        )

    ed = fire_edges(0)

    sc0 = jax.named_scope("hist_zero")
    sc0.__enter__()

    def zfill(i, _):
        for u in range(8):
            vbuf[pl.ds((i * 8 + u) * _L, _L)] = jnp.zeros((_L,), jnp.float32)
        return 0

    lax.fori_loop(0, _ZC // (8 * _L), zfill, 0)
    zdescs = []
    for j in range(_NZFULL // _NS):
        k = j * _NS + sid
        zdescs.append(pltpu.async_copy(
            vbuf.at[pl.ds(0, _ZC)], hist_sh.at[pl.ds(k * _ZC, _ZC)], zsem))

    @pl.when(sid < _NZFULL - (_NZFULL // _NS) * _NS)
    def _():
        k = (_NZFULL // _NS) * _NS + sid
        pltpu.sync_copy(vbuf.at[pl.ds(0, _ZC)],
                        hist_sh.at[pl.ds(k * _ZC, _ZC)])

    for i in range(_CHUNK // _L):
        ones_v[pl.ds(i * _L, _L)] = jnp.full((_L,), 1.0, jnp.float32)
    for d in zdescs:
        d.wait()

    plsc.subcore_barrier()
    sc0.__exit__(None, None, None)
    sc1 = jax.named_scope("hist_scatter")
    sc1.__enter__()

    sprev = []
    for t in range(len(steps)):
        g, _q = steps[t]
        e0, e1 = ebufs[t % 2]
        idx = ibufs[t % 2]
        ed[0].wait()
        ed[1].wait()
        if t + 1 < len(steps):
            ed = fire_edges(t + 1)
        g_off = g * _PLN

        def idx_row(r, _, e0=e0, e1=e1, idx=idx, g_off=g_off):
            for c in range(_CHUNK // _L):
                o = c * _L
                v0 = e0[pl.ds(r * _CHUNK + o, _L)]
                v1 = e1[pl.ds(r * _CHUNK + o, _L)]
                idx[r, pl.ds(o, _L)] = v0 * _W + v1 + g_off
            return 0

        lax.fori_loop(0, _NROW, idx_row, 0)
        for d in sprev:
            d.wait()
        sprev = [
            pltpu.async_copy(ones_v, hist_sh.at[idx.at[r]], ssem, add=True)
            for r in range(_NROW)
        ]
    for d in sprev:
        d.wait()

    plsc.subcore_barrier()
    sc1.__exit__(None, None, None)
    sc2 = jax.named_scope("hist_drain")
    sc2.__enter__()

    out_off = cid * _PSZ
    wr = [None, None]
    for j in range(_NDFULL // _NS):
        p = j % 2
        k = j * _NS + sid
        if wr[p] is not None:
            wr[p].wait()
        pltpu.sync_copy(hist_sh.at[pl.ds(k * _DC, _DC)],
                        vbuf.at[pl.ds(p * _DC, _DC)])
        wr[p] = pltpu.async_copy(vbuf.at[pl.ds(p * _DC, _DC)],
                                 out.at[pl.ds(out_off + k * _DC, _DC)], wsem)
    for d in wr:
        if d is not None:
            d.wait()

    @pl.when(sid < _NDFULL - (_NDFULL // _NS) * _NS)
    def _():
        k = (_NDFULL // _NS) * _NS + sid
        pltpu.sync_copy(hist_sh.at[pl.ds(k * _DC, _DC)],
                        vbuf.at[pl.ds(0, _DC)])
        pltpu.sync_copy(vbuf.at[pl.ds(0, _DC)],
                        out.at[pl.ds(out_off + k * _DC, _DC)])

    sc2.__exit__(None, None, None)


def _histogram3(e_gua, e_cos, e_fun):
    run = pl.kernel(
        _hist_body,
        out_type=jax.ShapeDtypeStruct((_NC * _PSZ,), jnp.float32),
        mesh=plsc.VectorSubcoreMesh(core_axis_name="c", subcore_axis_name="s"),
        scratch_types=[
            pltpu.VMEM((_ESH,), jnp.int32),
            pltpu.VMEM((_ESH,), jnp.int32),
            pltpu.VMEM((_ESH,), jnp.int32),
            pltpu.VMEM((_ESH,), jnp.int32),
            pltpu.VMEM((_NROW, _CHUNK), jnp.int32),
            pltpu.VMEM((_NROW, _CHUNK), jnp.int32),
            pltpu.VMEM((_CHUNK,), jnp.float32),
            pltpu.VMEM((_ZC,), jnp.float32),
            pltpu.VMEM_SHARED((_PSZ,), jnp.float32),
            pltpu.SemaphoreType.DMA,
            pltpu.SemaphoreType.DMA,
            pltpu.SemaphoreType.DMA,
            pltpu.SemaphoreType.DMA,
        ],
    )
    return run(e_gua, e_cos, e_fun).reshape(_NC, _PSZ // 128, 128)


def _conv_body(cnt_ref, mi_ref, x_ref, W1_ref, b1_ref, W2_ref, b2_ref, out_ref):
    F = cnt_ref[0] + cnt_ref[1]
    cnt = jnp.reshape(F, (N, _W))[:, 0:N]
    C = cnt * mi_ref[0]
    deg = jnp.sum(C, axis=0, keepdims=True) + 1.0
    r = lax.rsqrt(deg)
    r = r * (1.5 - 0.5 * deg * r * r)
    dis = jnp.where(deg > 0, r, 0.0)

    def layer(x, W, b, x_is_nd):
        cdims = (((0,), (1,)), ((), ())) if x_is_nd else (((0,), (0,)), ((), ()))
        hT = lax.dot_general(W, x, cdims, **_DOT)
        HdT = hT * dis
        aggT = lax.dot_general(HdT, C, (((1,), (0,)), ((), ())), **_DOTB) + HdT
        return jnp.maximum(dis * aggT + b, 0.0)

    x1T = layer(x_ref[...], W1_ref[...], b1_ref[...], True)
    x2T = layer(x1T, W2_ref[...], b2_ref[...], False)
    out_ref[0:D, :] = x1T
    out_ref[D:2 * D, :] = x2T


def _convs_tc(cnt2, mi3, x, W1, b1c, W2, b2c):
    return pl.pallas_call(
        _conv_body,
        grid=(3,),
        in_specs=[
            pl.BlockSpec((2, _PLN // 128, 128), lambda i: (0, i, 0)),
            pl.BlockSpec((1, N, N), lambda i: (i, 0, 0)),
            pl.BlockSpec((N, D), lambda i: (0, 0)),
            pl.BlockSpec((D, D), lambda i: (0, 0)),
            pl.BlockSpec((D, 1), lambda i: (0, 0)),
            pl.BlockSpec((D, D), lambda i: (0, 0)),
            pl.BlockSpec((D, 1), lambda i: (0, 0)),
        ],
        out_specs=pl.BlockSpec((2 * D, N), lambda i: (i, 0)),
        out_shape=jax.ShapeDtypeStruct((6 * D, N), jnp.float32),
    )(cnt2, mi3, x, W1, b1c, W2, b2c)


def _head_body(xm_ref, fc1_W_ref, fc1_b_ref, fc2_W_ref, fc2_b_ref,
               cnn_W_ref, cnn_b_ref, out_ref):
    xm = xm_ref[...]
    sums = [jnp.sum(xm[c]) for c in range(6)]
    att = jnp.stack(sums).reshape(1, 6) * (1.0 / (D * N))
    t1 = lax.dot_general(att, fc1_W_ref[...], (((1,), (1,)), ((), ())), **_DOT)
    t1 = jnp.maximum(t1 + fc1_b_ref[...], 0.0)
    t2 = lax.dot_general(t1, fc2_W_ref[...], (((1,), (1,)), ((), ())), **_DOT)
    t2 = t2 + fc2_b_ref[...]
    a = 1.0 / (1.0 + jnp.exp(-t2))
    coef = a * cnn_W_ref[...]
    acc = coef[0, 0] * xm[0]
    for c in range(1, 6):
        acc = acc + coef[0, c] * xm[c]
    out_ref[...] = acc + cnn_b_ref[0, 0]


def _head_tc(xm6, fc1_W, fc1_b, fc2_W, fc2_b, cnn_W, cnn_b):
    return pl.pallas_call(
        _head_body,
        out_shape=jax.ShapeDtypeStruct((D, N), jnp.float32),
    )(xm6, fc1_W, fc1_b, fc2_W, fc2_b, cnn_W, cnn_b)


def kernel(mi_gua, mi_gua_edges, mi_cos, mi_cos_edges, mi_fun, mi_fun_edges,
           mi_infeats, W1, b1, W2, b2, fc1_W, fc1_b, fc2_W, fc2_b, cnn_W, cnn_b):
    cnt2 = _histogram3(mi_gua_edges, mi_cos_edges, mi_fun_edges)
    mi3 = jnp.stack([mi_gua, mi_cos, mi_fun])
    XMT = _convs_tc(cnt2, mi3, mi_infeats,
                    W1, b1.reshape(D, 1), W2, b2.reshape(D, 1))
    xm6 = XMT.T.reshape(6, D, N)
    outT = _head_tc(xm6, fc1_W, fc1_b.reshape(1, 30),
                    fc2_W, fc2_b.reshape(1, 6),
                    cnn_W.reshape(1, 6), cnn_b.reshape(1, 1))
    return outT.T


# speedup vs baseline: 1.2056x; 1.0920x over previous
import jax
import jax.numpy as jnp
from jax import lax
from jax.experimental import pallas as pl
from jax.experimental.pallas import tpu as pltpu
from jax.experimental.pallas import tpu_sc as plsc

N = 728
D = 64
E = 262144

_DOT = dict(preferred_element_type=jnp.float32, precision=lax.Precision.HIGHEST)
_DOTB = dict(preferred_element_type=jnp.float32, precision=lax.Precision.DEFAULT)

_NC = 2
_NS = 16
_NW = _NC * _NS
_L = 16

_W = 768
_PLN = N * _W
_PSZ = 3 * _PLN
_EPW = E // _NW
_ESH = _EPW // 4
_CHUNK = 128
_NROW = _ESH // _CHUNK
_ZC = 4096
_DC = 2048
_NZFULL = _PSZ // _ZC
_ZTAIL = _PSZ - _NZFULL * _ZC
_NDFULL = _PSZ // _DC


def _hist_body(eg, ec, ef, out,
               e0a, e1a, e0b, e1b, idx_a, idx_b, ones_v, vbuf, hist_sh,
               zsem, esem, ssem, wsem):
    cid = lax.axis_index("c")
    sid = lax.axis_index("s")
    wid = sid * _NC + cid
    base = wid * _EPW
    edges = (eg, ec, ef)
    steps = [(g, q) for g in range(3) for q in range(4)]
    ebufs = ((e0a, e1a), (e0b, e1b))
    ibufs = (idx_a, idx_b)

    def fire_edges(t):
        g, q = steps[t]
        e0, e1 = ebufs[t % 2]
        off = base + q * _ESH
        return (
            pltpu.async_copy(edges[g].at[0, pl.ds(off, _ESH)], e0, esem),
            pltpu.async_copy(edges[g].at[1, pl.ds(off, _ESH)], e1, esem),
        )

    ed = fire_edges(0)

    sc0 = jax.named_scope("hist_zero")
    sc0.__enter__()

    def zfill(i, _):
        for u in range(8):
            vbuf[pl.ds((i * 8 + u) * _L, _L)] = jnp.zeros((_L,), jnp.float32)
        return 0

    lax.fori_loop(0, _ZC // (8 * _L), zfill, 0)
    zdescs = []
    for j in range(_NZFULL // _NS):
        k = j * _NS + sid
        zdescs.append(pltpu.async_copy(
            vbuf.at[pl.ds(0, _ZC)], hist_sh.at[pl.ds(k * _ZC, _ZC)], zsem))

    @pl.when(sid < _NZFULL - (_NZFULL // _NS) * _NS)
    def _():
        k = (_NZFULL // _NS) * _NS + sid
        pltpu.sync_copy(vbuf.at[pl.ds(0, _ZC)],
                        hist_sh.at[pl.ds(k * _ZC, _ZC)])

    @pl.when(sid == 0)
    def _():
        pltpu.sync_copy(vbuf.at[pl.ds(0, _ZTAIL)],
                        hist_sh.at[pl.ds(_NZFULL * _ZC, _ZTAIL)])

    for i in range(_CHUNK // _L):
        ones_v[pl.ds(i * _L, _L)] = jnp.full((_L,), 1.0, jnp.float32)
    for d in zdescs:
        d.wait()

    plsc.subcore_barrier()
    sc0.__exit__(None, None, None)
    sc1 = jax.named_scope("hist_scatter")
    sc1.__enter__()

    sprev = []
    for t in range(len(steps)):
        g, _q = steps[t]
        e0, e1 = ebufs[t % 2]
        idx = ibufs[t % 2]
        ed[0].wait()
        ed[1].wait()
        if t + 1 < len(steps):
            ed = fire_edges(t + 1)
        g_off = g * _PLN

        def idx_row(r, _, e0=e0, e1=e1, idx=idx, g_off=g_off):
            for c in range(_CHUNK // _L):
                o = c * _L
                v0 = e0[pl.ds(r * _CHUNK + o, _L)]
                v1 = e1[pl.ds(r * _CHUNK + o, _L)]
                idx[r, pl.ds(o, _L)] = v0 * _W + v1 + g_off
            return 0

        lax.fori_loop(0, _NROW, idx_row, 0)
        for d in sprev:
            d.wait()
        sprev = [
            pltpu.async_copy(ones_v, hist_sh.at[idx.at[r]], ssem, add=True)
            for r in range(_NROW)
        ]
    for d in sprev:
        d.wait()

    plsc.subcore_barrier()
    sc1.__exit__(None, None, None)
    sc2 = jax.named_scope("hist_drain")
    sc2.__enter__()

    out_off = cid * _PSZ
    wr = [None, None]
    for j in range(_NDFULL // _NS):
        p = j % 2
        k = j * _NS + sid
        if wr[p] is not None:
            wr[p].wait()
        pltpu.sync_copy(hist_sh.at[pl.ds(k * _DC, _DC)],
                        vbuf.at[pl.ds(p * _DC, _DC)])
        wr[p] = pltpu.async_copy(vbuf.at[pl.ds(p * _DC, _DC)],
                                 out.at[pl.ds(out_off + k * _DC, _DC)], wsem)
    for d in wr:
        if d is not None:
            d.wait()

    @pl.when(sid < _NDFULL - (_NDFULL // _NS) * _NS)
    def _():
        k = (_NDFULL // _NS) * _NS + sid
        pltpu.sync_copy(hist_sh.at[pl.ds(k * _DC, _DC)],
                        vbuf.at[pl.ds(0, _DC)])
        pltpu.sync_copy(vbuf.at[pl.ds(0, _DC)],
                        out.at[pl.ds(out_off + k * _DC, _DC)])

    sc2.__exit__(None, None, None)


def _histogram3(e_gua, e_cos, e_fun):
    run = pl.kernel(
        _hist_body,
        out_type=jax.ShapeDtypeStruct((_NC * _PSZ,), jnp.float32),
        mesh=plsc.VectorSubcoreMesh(core_axis_name="c", subcore_axis_name="s"),
        scratch_types=[
            pltpu.VMEM((_ESH,), jnp.int32),
            pltpu.VMEM((_ESH,), jnp.int32),
            pltpu.VMEM((_ESH,), jnp.int32),
            pltpu.VMEM((_ESH,), jnp.int32),
            pltpu.VMEM((_NROW, _CHUNK), jnp.int32),
            pltpu.VMEM((_NROW, _CHUNK), jnp.int32),
            pltpu.VMEM((_CHUNK,), jnp.float32),
            pltpu.VMEM((_ZC,), jnp.float32),
            pltpu.VMEM_SHARED((_PSZ,), jnp.float32),
            pltpu.SemaphoreType.DMA,
            pltpu.SemaphoreType.DMA,
            pltpu.SemaphoreType.DMA,
            pltpu.SemaphoreType.DMA,
        ],
    )
    return run(e_gua, e_cos, e_fun).reshape(_NC, _PSZ // 128, 128)


def _conv_body(cnt_ref, mi_ref, x_ref, W1_ref, b1_ref, W2_ref, b2_ref, out_ref):
    F = cnt_ref[0] + cnt_ref[1]
    cnt = jnp.reshape(F, (N, _W))[:, 0:N]
    C = cnt * mi_ref[0]
    deg = jnp.sum(C, axis=0, keepdims=True) + 1.0
    r = lax.rsqrt(deg)
    r = r * (1.5 - 0.5 * deg * r * r)
    dis = jnp.where(deg > 0, r, 0.0)

    def layer(x, W, b, x_is_nd):
        cdims = (((0,), (1,)), ((), ())) if x_is_nd else (((0,), (0,)), ((), ()))
        hT = lax.dot_general(W, x, cdims, **_DOT)
        HdT = hT * dis
        aggT = lax.dot_general(HdT, C, (((1,), (0,)), ((), ())), **_DOTB) + HdT
        return jnp.maximum(dis * aggT + b, 0.0)

    x1T = layer(x_ref[...], W1_ref[...], b1_ref[...], True)
    x2T = layer(x1T, W2_ref[...], b2_ref[...], False)
    out_ref[0:D, :] = x1T
    out_ref[D:2 * D, :] = x2T


def _convs_tc(cnt2, mi3, x, W1, b1c, W2, b2c):
    return pl.pallas_call(
        _conv_body,
        grid=(3,),
        in_specs=[
            pl.BlockSpec((2, _PLN // 128, 128), lambda i: (0, i, 0)),
            pl.BlockSpec((1, N, N), lambda i: (i, 0, 0)),
            pl.BlockSpec((N, D), lambda i: (0, 0)),
            pl.BlockSpec((D, D), lambda i: (0, 0)),
            pl.BlockSpec((D, 1), lambda i: (0, 0)),
            pl.BlockSpec((D, D), lambda i: (0, 0)),
            pl.BlockSpec((D, 1), lambda i: (0, 0)),
        ],
        out_specs=pl.BlockSpec((2 * D, N), lambda i: (i, 0)),
        out_shape=jax.ShapeDtypeStruct((6 * D, N), jnp.float32),
    )(cnt2, mi3, x, W1, b1c, W2, b2c)


def _head_body(xm_ref, fc1_W_ref, fc1_b_ref, fc2_W_ref, fc2_b_ref,
               cnn_W_ref, cnn_b_ref, out_ref):
    xm = xm_ref[...]
    sums = [jnp.sum(xm[c]) for c in range(6)]
    att = jnp.stack(sums).reshape(1, 6) * (1.0 / (D * N))
    t1 = lax.dot_general(att, fc1_W_ref[...], (((1,), (1,)), ((), ())), **_DOT)
    t1 = jnp.maximum(t1 + fc1_b_ref[...], 0.0)
    t2 = lax.dot_general(t1, fc2_W_ref[...], (((1,), (1,)), ((), ())), **_DOT)
    t2 = t2 + fc2_b_ref[...]
    a = 1.0 / (1.0 + jnp.exp(-t2))
    coef = a * cnn_W_ref[...]
    acc = coef[0, 0] * xm[0]
    for c in range(1, 6):
        acc = acc + coef[0, c] * xm[c]
    out_ref[...] = acc + cnn_b_ref[0, 0]


def _head_tc(xm6, fc1_W, fc1_b, fc2_W, fc2_b, cnn_W, cnn_b):
    return pl.pallas_call(
        _head_body,
        out_shape=jax.ShapeDtypeStruct((D, N), jnp.float32),
    )(xm6, fc1_W, fc1_b, fc2_W, fc2_b, cnn_W, cnn_b)


def kernel(mi_gua, mi_gua_edges, mi_cos, mi_cos_edges, mi_fun, mi_fun_edges,
           mi_infeats, W1, b1, W2, b2, fc1_W, fc1_b, fc2_W, fc2_b, cnn_W, cnn_b):
    cnt2 = _histogram3(mi_gua_edges, mi_cos_edges, mi_fun_edges)
    mi3 = jnp.stack([mi_gua, mi_cos, mi_fun])
    XMT = _convs_tc(cnt2, mi3, mi_infeats,
                    W1, b1.reshape(D, 1), W2, b2.reshape(D, 1))
    xm6 = XMT.T.reshape(6, D, N)
    outT = _head_tc(xm6, fc1_W, fc1_b.reshape(1, 30),
                    fc2_W, fc2_b.reshape(1, 6),
                    cnn_W.reshape(1, 6), cnn_b.reshape(1, 1))
    return outT.T


# speedup vs baseline: 1.2060x; 1.0003x over previous
import jax
import jax.numpy as jnp
from jax import lax
from jax.experimental import pallas as pl
from jax.experimental.pallas import tpu as pltpu
from jax.experimental.pallas import tpu_sc as plsc

N = 728
D = 64
E = 262144

_DOT = dict(preferred_element_type=jnp.float32, precision=lax.Precision.HIGHEST)
_DOTB = dict(preferred_element_type=jnp.float32, precision=lax.Precision.DEFAULT)

_NC = 2
_NS = 16
_NW = _NC * _NS
_L = 16

_W = 768
_PLN = N * _W
_PSZ = 3 * _PLN
_EPW = E // _NW
_ESH = _EPW // 4
_CHUNK = 128
_NROW = _ESH // _CHUNK
_ZC = 4096
_DC = 2048
_NZFULL = _PSZ // _ZC
_ZTAIL = _PSZ - _NZFULL * _ZC
_NDFULL = _PSZ // _DC


def _hist_body(eg, ec, ef, out,
               e0a, e1a, e0b, e1b, idx_a, idx_b, ones_v, vbuf, hist_sh,
               zsem, esem, ssem, wsem):
    cid = lax.axis_index("c")
    sid = lax.axis_index("s")
    wid = sid * _NC + cid
    base = wid * _EPW
    edges = (eg, ec, ef)
    steps = [(g, q) for g in range(3) for q in range(4)]
    ebufs = ((e0a, e1a), (e0b, e1b))
    ibufs = (idx_a, idx_b)

    def fire_edges(t):
        g, q = steps[t]
        e0, e1 = ebufs[t % 2]
        off = base + q * _ESH
        return (
            pltpu.async_copy(edges[g].at[0, pl.ds(off, _ESH)], e0, esem),
            pltpu.async_copy(edges[g].at[1, pl.ds(off, _ESH)], e1, esem),
        )

    ed = fire_edges(0)

    def zfill(i, _):
        for u in range(8):
            vbuf[pl.ds((i * 8 + u) * _L, _L)] = jnp.zeros((_L,), jnp.float32)
        return 0

    lax.fori_loop(0, _ZC // (8 * _L), zfill, 0)
    zdescs = []
    for j in range(_NZFULL // _NS):
        k = j * _NS + sid
        zdescs.append(pltpu.async_copy(
            vbuf.at[pl.ds(0, _ZC)], hist_sh.at[pl.ds(k * _ZC, _ZC)], zsem))

    @pl.when(sid < _NZFULL - (_NZFULL // _NS) * _NS)
    def _():
        k = (_NZFULL // _NS) * _NS + sid
        pltpu.sync_copy(vbuf.at[pl.ds(0, _ZC)],
                        hist_sh.at[pl.ds(k * _ZC, _ZC)])

    @pl.when(sid == 0)
    def _():
        pltpu.sync_copy(vbuf.at[pl.ds(0, _ZTAIL)],
                        hist_sh.at[pl.ds(_NZFULL * _ZC, _ZTAIL)])

    for i in range(_CHUNK // _L):
        ones_v[pl.ds(i * _L, _L)] = jnp.full((_L,), 1.0, jnp.float32)
    for d in zdescs:
        d.wait()

    plsc.subcore_barrier()

    sprev = []
    for t in range(len(steps)):
        g, _q = steps[t]
        e0, e1 = ebufs[t % 2]
        idx = ibufs[t % 2]
        ed[0].wait()
        ed[1].wait()
        if t + 1 < len(steps):
            ed = fire_edges(t + 1)
        g_off = g * _PLN

        def idx_row(r, _, e0=e0, e1=e1, idx=idx, g_off=g_off):
            for c in range(_CHUNK // _L):
                o = c * _L
                v0 = e0[pl.ds(r * _CHUNK + o, _L)]
                v1 = e1[pl.ds(r * _CHUNK + o, _L)]
                idx[r, pl.ds(o, _L)] = v0 * _W + v1 + g_off
            return 0

        lax.fori_loop(0, _NROW, idx_row, 0)
        for d in sprev:
            d.wait()
        sprev = [
            pltpu.async_copy(ones_v, hist_sh.at[idx.at[r]], ssem, add=True)
            for r in range(_NROW)
        ]
    for d in sprev:
        d.wait()

    plsc.subcore_barrier()

    out_off = cid * _PSZ
    wr = [None, None]
    for j in range(_NDFULL // _NS):
        p = j % 2
        k = j * _NS + sid
        if wr[p] is not None:
            wr[p].wait()
        pltpu.sync_copy(hist_sh.at[pl.ds(k * _DC, _DC)],
                        vbuf.at[pl.ds(p * _DC, _DC)])
        wr[p] = pltpu.async_copy(vbuf.at[pl.ds(p * _DC, _DC)],
                                 out.at[pl.ds(out_off + k * _DC, _DC)], wsem)
    for d in wr:
        if d is not None:
            d.wait()

    @pl.when(sid < _NDFULL - (_NDFULL // _NS) * _NS)
    def _():
        k = (_NDFULL // _NS) * _NS + sid
        pltpu.sync_copy(hist_sh.at[pl.ds(k * _DC, _DC)],
                        vbuf.at[pl.ds(0, _DC)])
        pltpu.sync_copy(vbuf.at[pl.ds(0, _DC)],
                        out.at[pl.ds(out_off + k * _DC, _DC)])


def _histogram3(e_gua, e_cos, e_fun):
    run = pl.kernel(
        _hist_body,
        out_type=jax.ShapeDtypeStruct((_NC * _PSZ,), jnp.float32),
        mesh=plsc.VectorSubcoreMesh(core_axis_name="c", subcore_axis_name="s"),
        scratch_types=[
            pltpu.VMEM((_ESH,), jnp.int32),
            pltpu.VMEM((_ESH,), jnp.int32),
            pltpu.VMEM((_ESH,), jnp.int32),
            pltpu.VMEM((_ESH,), jnp.int32),
            pltpu.VMEM((_NROW, _CHUNK), jnp.int32),
            pltpu.VMEM((_NROW, _CHUNK), jnp.int32),
            pltpu.VMEM((_CHUNK,), jnp.float32),
            pltpu.VMEM((_ZC,), jnp.float32),
            pltpu.VMEM_SHARED((_PSZ,), jnp.float32),
            pltpu.SemaphoreType.DMA,
            pltpu.SemaphoreType.DMA,
            pltpu.SemaphoreType.DMA,
            pltpu.SemaphoreType.DMA,
        ],
    )
    return run(e_gua, e_cos, e_fun).reshape(_NC, _PSZ // 128, 128)


def _conv_body(cnt_ref, mi_ref, x_ref, W1_ref, b1_ref, W2_ref, b2_ref, out_ref):
    F = cnt_ref[0] + cnt_ref[1]
    cnt = jnp.reshape(F, (N, _W))[:, 0:N]
    C = cnt * mi_ref[0]
    deg = jnp.sum(C, axis=0, keepdims=True) + 1.0
    r = lax.rsqrt(deg)
    r = r * (1.5 - 0.5 * deg * r * r)
    dis = jnp.where(deg > 0, r, 0.0)

    def layer(x, W, b, x_is_nd):
        cdims = (((0,), (1,)), ((), ())) if x_is_nd else (((0,), (0,)), ((), ()))
        hT = lax.dot_general(W, x, cdims, **_DOT)
        HdT = hT * dis
        aggT = lax.dot_general(HdT, C, (((1,), (0,)), ((), ())), **_DOTB) + HdT
        return jnp.maximum(dis * aggT + b, 0.0)

    x1T = layer(x_ref[...], W1_ref[...], b1_ref[...], True)
    x2T = layer(x1T, W2_ref[...], b2_ref[...], False)
    out_ref[0:D, :] = x1T
    out_ref[D:2 * D, :] = x2T


def _convs_tc(cnt2, mi3, x, W1, b1c, W2, b2c):
    return pl.pallas_call(
        _conv_body,
        grid=(3,),
        in_specs=[
            pl.BlockSpec((2, _PLN // 128, 128), lambda i: (0, i, 0)),
            pl.BlockSpec((1, N, N), lambda i: (i, 0, 0)),
            pl.BlockSpec((N, D), lambda i: (0, 0)),
            pl.BlockSpec((D, D), lambda i: (0, 0)),
            pl.BlockSpec((D, 1), lambda i: (0, 0)),
            pl.BlockSpec((D, D), lambda i: (0, 0)),
            pl.BlockSpec((D, 1), lambda i: (0, 0)),
        ],
        out_specs=pl.BlockSpec((2 * D, N), lambda i: (i, 0)),
        out_shape=jax.ShapeDtypeStruct((6 * D, N), jnp.float32),
    )(cnt2, mi3, x, W1, b1c, W2, b2c)


def _head_body(xm_ref, fc1_W_ref, fc1_b_ref, fc2_W_ref, fc2_b_ref,
               cnn_W_ref, cnn_b_ref, out_ref):
    xm = xm_ref[...]
    sums = [jnp.sum(xm[c]) for c in range(6)]
    att = jnp.stack(sums).reshape(1, 6) * (1.0 / (D * N))
    t1 = lax.dot_general(att, fc1_W_ref[...], (((1,), (1,)), ((), ())), **_DOT)
    t1 = jnp.maximum(t1 + fc1_b_ref[...], 0.0)
    t2 = lax.dot_general(t1, fc2_W_ref[...], (((1,), (1,)), ((), ())), **_DOT)
    t2 = t2 + fc2_b_ref[...]
    a = 1.0 / (1.0 + jnp.exp(-t2))
    coef = a * cnn_W_ref[...]
    acc = coef[0, 0] * xm[0]
    for c in range(1, 6):
        acc = acc + coef[0, c] * xm[c]
    out_ref[...] = acc + cnn_b_ref[0, 0]


def _head_tc(xm6, fc1_W, fc1_b, fc2_W, fc2_b, cnn_W, cnn_b):
    return pl.pallas_call(
        _head_body,
        out_shape=jax.ShapeDtypeStruct((D, N), jnp.float32),
    )(xm6, fc1_W, fc1_b, fc2_W, fc2_b, cnn_W, cnn_b)


def kernel(mi_gua, mi_gua_edges, mi_cos, mi_cos_edges, mi_fun, mi_fun_edges,
           mi_infeats, W1, b1, W2, b2, fc1_W, fc1_b, fc2_W, fc2_b, cnn_W, cnn_b):
    cnt2 = _histogram3(mi_gua_edges, mi_cos_edges, mi_fun_edges)
    mi3 = jnp.stack([mi_gua, mi_cos, mi_fun])
    XMT = _convs_tc(cnt2, mi3, mi_infeats,
                    W1, b1.reshape(D, 1), W2, b2.reshape(D, 1))
    xm6 = XMT.T.reshape(6, D, N)
    outT = _head_tc(xm6, fc1_W, fc1_b.reshape(1, 30),
                    fc2_W, fc2_b.reshape(1, 6),
                    cnn_W.reshape(1, 6), cnn_b.reshape(1, 1))
    return outT.T
